# Initial kernel scaffold; baseline (speedup 1.0000x reference)
import jax
import jax.numpy as jnp
from jax.experimental import pallas as pl


def kernel(x, gate_W, gate_b, W1, b1, W2, b2):
    raise NotImplementedError("write your pallas kernel here")


# speedup vs baseline: 1.0949x; 1.0949x over previous
import functools

import jax
import jax.numpy as jnp
from jax import lax
from jax.experimental import pallas as pl
from jax.experimental.pallas import tpu as pltpu

_E = 8
_K = 2
_BT = 256


def _ffn_kernel(te_ref, xs_ref, w1_ref, b1_ref, w2_ref, b2_ref, out_ref):
    e = te_ref[pl.program_id(0)]
    x = xs_ref[...]
    h = jnp.dot(x, w1_ref[0], preferred_element_type=jnp.float32)
    h = jnp.maximum(h + b1_ref[e][None, :], 0.0)
    y = jnp.dot(h, w2_ref[0], preferred_element_type=jnp.float32)
    out_ref[...] = y + b2_ref[e][None, :]


def _grouped_ffn(xs, tile_expert, W1, b1, W2, b2, nt, d):
    grid_spec = pltpu.PrefetchScalarGridSpec(
        num_scalar_prefetch=1,
        grid=(nt,),
        in_specs=[
            pl.BlockSpec((_BT, d), lambda i, te: (i, 0)),
            pl.BlockSpec((1, d, d), lambda i, te: (te[i], 0, 0)),
            pl.BlockSpec((_E, d), lambda i, te: (0, 0)),
            pl.BlockSpec((1, d, d), lambda i, te: (te[i], 0, 0)),
            pl.BlockSpec((_E, d), lambda i, te: (0, 0)),
        ],
        out_specs=pl.BlockSpec((_BT, d), lambda i, te: (i, 0)),
    )
    return pl.pallas_call(
        _ffn_kernel,
        grid_spec=grid_spec,
        out_shape=jax.ShapeDtypeStruct((nt * _BT, d), jnp.float32),
    )(tile_expert, xs, W1, b1, W2, b2)


def kernel(x, gate_W, gate_b, W1, b1, W2, b2):
    x_shape = x.shape
    d = x_shape[-1]
    xf = x.reshape(-1, d)
    t = xf.shape[0]
    nt = (t * _K) // _BT + _E
    ntot = nt * _BT

    logits = xf @ gate_W + gate_b
    prob = jax.nn.softmax(logits, axis=-1)
    top_w, top_i = lax.top_k(prob, _K)
    wn = jax.nn.softmax(top_w, axis=-1)

    flat_e = jnp.concatenate([top_i[:, 0], top_i[:, 1]])
    oh = jax.nn.one_hot(flat_e, _E, dtype=jnp.int32)
    counts = jnp.sum(oh, axis=0)
    padded = ((counts + _BT - 1) // _BT) * _BT
    pad_off = jnp.concatenate(
        [jnp.zeros((1,), jnp.int32), jnp.cumsum(padded)[:-1].astype(jnp.int32)]
    )
    rank = jnp.cumsum(oh, axis=0) - oh
    rank_j = jnp.take_along_axis(rank, flat_e[:, None], axis=1)[:, 0]
    pos = pad_off[flat_e] + rank_j
    p1, p2 = pos[:t], pos[t:]

    tok = jnp.concatenate([jnp.arange(t, dtype=jnp.int32)] * _K)
    gather_tok = jnp.zeros((ntot,), jnp.int32).at[pos].set(tok)

    pad_end = pad_off + padded
    tile_start = jnp.arange(nt, dtype=jnp.int32) * _BT
    tile_expert = jnp.sum(
        (pad_end[None, :] <= tile_start[:, None]).astype(jnp.int32), axis=1
    )
    tile_expert = jnp.minimum(tile_expert, _E - 1)

    xs = xf[gather_tok]

    ys = _grouped_ffn(xs, tile_expert, W1, b1, W2, b2, nt, d)

    y = wn[:, 0:1] * ys[p1] + wn[:, 1:2] * ys[p2]
    return (y.reshape(x_shape), prob)
